# baseline (speedup 1.0000x reference)
import jax
import jax.numpy as jnp
from jax.experimental import pallas as pl
from jax.experimental.pallas import tpu as pltpu

H, W, C, KH, KW = 32, 32, 32, 4, 4
B, XH, XW = 16, 16, 16
D = C * KH * KW
NQ = H * W
NR = XH - KH + 1
M = NR * XW
XROWS = XH * XW + KH * XW


def _som_kernel(xt_ref, codes_ref, out_ref, c2_ref):
    @pl.when(pl.program_id(0) == 0)
    def _():
        codes0 = codes_ref[...]
        c2_ref[...] = jnp.sum(codes0 * codes0, axis=0, keepdims=True)

    parts = [
        xt_ref[0, kh * XW + kw:kh * XW + kw + M, :]
        for kh in range(KH)
        for kw in range(KW)
    ]
    p = jnp.concatenate(parts, axis=1)
    p2 = jnp.sum(p * p, axis=1, keepdims=True)
    cross = jax.lax.dot_general(
        p, codes_ref[...], (((1,), (0,)), ((), ())),
        preferred_element_type=jnp.float32)
    mse = (p2 - 2.0 * cross + c2_ref[...]) / D
    idx = jnp.argmin(mse, axis=1)
    wr = (idx // W).astype(jnp.float32) / H
    wc = (idx % W).astype(jnp.float32) / W
    out_ref[0, :, pl.ds(0, M)] = jnp.stack([wr, wc], axis=0)


@jax.jit
def kernel(x, som):
    xt = x.transpose(0, 2, 3, 1).reshape(B, XH * XW, C)
    xt = jnp.pad(xt, ((0, 0), (0, KH * XW), (0, 0)))
    codes = som.transpose(3, 4, 2, 0, 1).reshape(D, NQ)
    out = pl.pallas_call(
        _som_kernel,
        grid=(B,),
        in_specs=[
            pl.BlockSpec((1, XROWS, C), lambda b: (b, 0, 0)),
            pl.BlockSpec((D, NQ), lambda b: (0, 0)),
        ],
        out_specs=pl.BlockSpec((1, 2, XH * XW), lambda b: (b, 0, 0)),
        out_shape=jax.ShapeDtypeStruct((B, 2, XH * XW), jnp.float32),
        scratch_shapes=[pltpu.VMEM((1, NQ), jnp.float32)],
    )(xt, codes)
    out = out.reshape(B, 2, XH, XW)[:, :, :NR, :NR]
    return out
